# baseline (speedup 1.0000x reference)
import jax
import jax.numpy as jnp
from jax.experimental import pallas as pl
from jax.experimental.pallas import tpu as pltpu

_BM = 256
_DEPTH = 2


def _body(x_ref, a_hbm, o_ref, abuf0, abuf1, sem0, sem1):
    i = pl.program_id(0)
    steps = pl.num_programs(0)
    nbuf = 2 * _DEPTH

    @pl.when(i == 0)
    def _prologue():
        for j in range(nbuf):
            buf, sem = (abuf0, sem0) if j % 2 == 0 else (abuf1, sem1)
            pltpu.make_async_copy(
                a_hbm.at[pl.ds(j * _BM, _BM), :], buf.at[j // 2], sem.at[j // 2]
            ).start(priority=j % 2)

    slot = jax.lax.rem(jax.lax.div(i, 2), _DEPTH)

    @pl.when(jax.lax.rem(i, 2) == 0)
    def _even():
        pltpu.make_async_copy(
            a_hbm.at[pl.ds(i * _BM, _BM), :], abuf0.at[slot], sem0.at[slot]
        ).wait()
        t = jnp.dot(abuf0[slot], x_ref[...], preferred_element_type=jnp.float32)
        o_ref[...] = (t > 0.5).astype(jnp.float32)

        @pl.when(i + nbuf < steps)
        def _():
            nxt = i + nbuf
            pltpu.make_async_copy(
                a_hbm.at[pl.ds(nxt * _BM, _BM), :], abuf0.at[slot], sem0.at[slot]
            ).start()

    @pl.when(jax.lax.rem(i, 2) == 1)
    def _odd():
        pltpu.make_async_copy(
            a_hbm.at[pl.ds(i * _BM, _BM), :], abuf1.at[slot], sem1.at[slot]
        ).wait()
        t = jnp.dot(abuf1[slot], x_ref[...], preferred_element_type=jnp.float32)
        o_ref[...] = (t > 0.5).astype(jnp.float32)

        @pl.when(i + nbuf < steps)
        def _():
            nxt = i + nbuf
            pltpu.make_async_copy(
                a_hbm.at[pl.ds(nxt * _BM, _BM), :], abuf1.at[slot], sem1.at[slot]
            ).start(priority=1)


def kernel(x, a):
    m, k = a.shape
    n = x.shape[1]
    return pl.pallas_call(
        _body,
        grid=(m // _BM,),
        in_specs=[
            pl.BlockSpec((k, n), lambda i: (0, 0)),
            pl.BlockSpec(memory_space=pltpu.MemorySpace.HBM),
        ],
        out_specs=pl.BlockSpec((_BM, n), lambda i: (i, 0)),
        out_shape=jax.ShapeDtypeStruct((m, n), jnp.float32),
        scratch_shapes=[
            pltpu.VMEM((_DEPTH, _BM, 8192), jnp.float32),
            pltpu.VMEM((_DEPTH, _BM, 8192), jnp.float32),
            pltpu.SemaphoreType.DMA((_DEPTH,)),
            pltpu.SemaphoreType.DMA((_DEPTH,)),
        ],
        compiler_params=pltpu.CompilerParams(
            dimension_semantics=("arbitrary",),
        ),
    )(x, a)


# speedup vs baseline: 1.0074x; 1.0074x over previous
import jax
import jax.numpy as jnp
from jax.experimental import pallas as pl
from jax.experimental.pallas import tpu as pltpu

_BM = 256
_NBUF = 4


def _outer(x_ref, a_hbm, o_hbm):
    def body(a_blk, o_blk):
        t = jnp.dot(a_blk[...], x_ref[...], preferred_element_type=jnp.float32)
        o_blk[...] = (t > 0.5).astype(jnp.float32)

    m = a_hbm.shape[0]
    k = a_hbm.shape[1]
    n = o_hbm.shape[1]
    pltpu.emit_pipeline(
        body,
        grid=(m // _BM,),
        in_specs=[
            pl.BlockSpec(
                (_BM, k),
                lambda i: (i, 0),
                pipeline_mode=pl.Buffered(buffer_count=_NBUF),
            ),
        ],
        out_specs=[pl.BlockSpec((_BM, n), lambda i: (i, 0))],
    )(a_hbm, o_hbm)


def kernel(x, a):
    m, k = a.shape
    n = x.shape[1]
    return pl.pallas_call(
        _outer,
        in_specs=[
            pl.BlockSpec((k, n), lambda: (0, 0)),
            pl.BlockSpec(memory_space=pltpu.MemorySpace.HBM),
        ],
        out_specs=pl.BlockSpec(memory_space=pltpu.MemorySpace.HBM),
        out_shape=jax.ShapeDtypeStruct((m, n), jnp.float32),
    )(x, a)


# speedup vs baseline: 1.0561x; 1.0484x over previous
import jax
import jax.numpy as jnp
from jax.experimental import pallas as pl
from jax.experimental.pallas import tpu as pltpu

_BM = 256


def _fused_block(x_ref, a_ref, o_ref):
    t = jnp.dot(a_ref[...], x_ref[...], preferred_element_type=jnp.float32)
    o_ref[...] = (t > 0.5).astype(jnp.float32)


def kernel(x, a):
    m, k = a.shape
    n = x.shape[1]
    return pl.pallas_call(
        _fused_block,
        grid=(m // _BM,),
        in_specs=[
            pl.BlockSpec((k, n), lambda i: (0, 0)),
            pl.BlockSpec((_BM, k), lambda i: (i, 0)),
        ],
        out_specs=pl.BlockSpec((_BM, n), lambda i: (i, 0)),
        out_shape=jax.ShapeDtypeStruct((m, n), jnp.float32),
        compiler_params=pltpu.CompilerParams(
            dimension_semantics=("arbitrary",),
        ),
    )(x, a)
